# Initial kernel scaffold; baseline (speedup 1.0000x reference)
import jax
import jax.numpy as jnp
from jax.experimental import pallas as pl


def kernel(X, edge_index, edge_weight, W_cz, b_cz, W_cr, b_cr, W_ch, b_ch, Wz, bz, Wr, br, Wh, bh, W_fc, b_fc):
    raise NotImplementedError("write your pallas kernel here")


# speedup vs baseline: 7.9381x; 7.9381x over previous
import jax
import jax.numpy as jnp
from jax import lax
from jax.experimental import pallas as pl
from jax.experimental.pallas import tpu as pltpu
from jax.experimental.pallas import tpu_sc as plsc

_N = 10000
_E = 320000
_F = 128
_HID = 64
_P = 12
_NC = 2
_NS = 16
_NW = _NC * _NS
_EPW = _E // _NW
_CHUNK = 80
_NCHUNK = _EPW // _CHUNK
_RPT = _N // _NS
_ZR = 25
_DEG_B = 5
_DEG_R = 50
_DEG_C = 80
_LN = 16
_DPT = 640
_NPAD = _NS * _DPT
_CR = 632
_CRL_OFF = (_NS - 1) * _CR
_CRL = _N - _CRL_OFF

_MESH = plsc.VectorSubcoreMesh(core_axis_name="c", subcore_axis_name="s",
                               num_cores=_NC, num_subcores=_NS)


def _deg_body(dstdeg_hbm, ewdeg_hbm, deg_hbm, deg_sh, ddst, dew, ones_b):
    cid = lax.axis_index("c")
    sid = lax.axis_index("s")

    ov = jnp.ones((_LN,), jnp.float32)

    def _fill_o(i, carry):
        ones_b[pl.ds(i * _LN, _LN)] = ov
        return carry

    lax.fori_loop(0, _DPT // _LN, _fill_o, 0)

    pltpu.sync_copy(ones_b, deg_sh.at[pl.ds(sid * _DPT, _DPT)])
    plsc.subcore_barrier()

    for b in range(_DEG_B):
        pltpu.sync_copy(dstdeg_hbm.at[sid, b], ddst)
        pltpu.sync_copy(ewdeg_hbm.at[sid, b], dew)

        def _deg_scatter(j, carry):
            pltpu.sync_copy(dew.at[j], deg_sh.at[ddst.at[j]], add=True)
            return carry

        lax.fori_loop(0, _DEG_R, _deg_scatter, 0)
    plsc.subcore_barrier()

    @pl.when(cid == 0)
    def _():
        pltpu.sync_copy(deg_sh.at[pl.ds(sid * _DPT, _DPT)],
                        deg_hbm.at[pl.ds(sid * _DPT, _DPT)])


_deg_call = pl.kernel(
    _deg_body,
    out_type=[jax.ShapeDtypeStruct((_NPAD,), jnp.float32)],
    mesh=_MESH,
    scratch_types=[
        pltpu.VMEM_SHARED((_NPAD,), jnp.float32),
        pltpu.VMEM((_DEG_R, _DEG_C), jnp.int32),
        pltpu.VMEM((_DEG_R, _DEG_C), jnp.float32),
        pltpu.VMEM((_DPT,), jnp.float32),
    ],
)


def _agg_body(xs_hbm, src_hbm, dst_hbm, ew_hbm, y0_hbm, y1_hbm,
              y_sh, src_b, dst_b, ew_b, rows, zb, sem):
    cid = lax.axis_index("c")
    sid = lax.axis_index("s")
    wid = cid * _NS + sid

    zv = jnp.zeros((_LN,), jnp.float32)

    def _fill_z(i, carry):
        r = i // (_F // _LN)
        j = i % (_F // _LN)
        zb[r, pl.ds(j * _LN, _LN)] = zv
        return carry

    lax.fori_loop(0, _ZR * (_F // _LN), _fill_z, 0)

    pltpu.sync_copy(src_hbm.at[wid], src_b)
    pltpu.sync_copy(dst_hbm.at[wid], dst_b)
    pltpu.sync_copy(ew_hbm.at[wid], ew_b)

    def _p_body(p, carry):
        for k in range(_RPT // _ZR):
            pltpu.sync_copy(zb, y_sh.at[pl.ds(sid * _RPT + k * _ZR, _ZR)])
        plsc.subcore_barrier()

        def _chunk(c, ccarry):
            pltpu.async_copy(
                xs_hbm.at[src_b.at[pl.ds(c * _CHUNK, _CHUNK)]], rows,
                sem).wait()
            for g5 in range(_CHUNK // _LN):
                ew16 = ew_b[pl.ds(c * _CHUNK + g5 * _LN, _LN)]
                for t in range(_LN):
                    nv = ew16[t]
                    g = g5 * _LN + t
                    for j in range(_F // _LN):
                        rows[g, pl.ds(j * _LN, _LN)] = (
                            rows[g, pl.ds(j * _LN, _LN)] * nv)
            pltpu.sync_copy(rows, y_sh.at[dst_b.at[c]], add=True)
            return ccarry

        lax.fori_loop(0, _NCHUNK, _chunk, 0)
        plsc.subcore_barrier()

        @pl.when(cid == 0)
        def _():
            @pl.when(sid < _NS - 1)
            def _():
                pltpu.sync_copy(y_sh.at[pl.ds(sid * _CR, _CR)],
                                y0_hbm.at[p, pl.ds(sid * _CR, _CR)])

            @pl.when(sid == _NS - 1)
            def _():
                pltpu.sync_copy(y_sh.at[pl.ds(_CRL_OFF, _CRL)],
                                y0_hbm.at[p, pl.ds(_CRL_OFF, _CRL)])

        @pl.when(cid == 1)
        def _():
            @pl.when(sid < _NS - 1)
            def _():
                pltpu.sync_copy(y_sh.at[pl.ds(sid * _CR, _CR)],
                                y1_hbm.at[p, pl.ds(sid * _CR, _CR)])

            @pl.when(sid == _NS - 1)
            def _():
                pltpu.sync_copy(y_sh.at[pl.ds(_CRL_OFF, _CRL)],
                                y1_hbm.at[p, pl.ds(_CRL_OFF, _CRL)])

        plsc.subcore_barrier()

        def _adv(i, acarry):
            src_b[pl.ds(i * _LN, _LN)] = (
                src_b[pl.ds(i * _LN, _LN)] + jnp.int32(_N))
            return acarry

        lax.fori_loop(0, _EPW // _LN, _adv, 0)
        return carry

    lax.fori_loop(0, _P, _p_body, 0)


_agg_call = pl.kernel(
    _agg_body,
    out_type=[
        jax.ShapeDtypeStruct((_P, _N, _F), jnp.float32),
        jax.ShapeDtypeStruct((_P, _N, _F), jnp.float32),
    ],
    mesh=_MESH,
    scratch_types=[
        pltpu.VMEM_SHARED((_N, _F), jnp.float32),
        pltpu.VMEM((_EPW,), jnp.int32),
        pltpu.VMEM((_NCHUNK, _CHUNK), jnp.int32),
        pltpu.VMEM((_EPW,), jnp.float32),
        pltpu.VMEM((_CHUNK, _F), jnp.float32),
        pltpu.VMEM((_ZR, _F), jnp.float32),
        pltpu.SemaphoreType.DMA,
    ],
)

_BN = 2000
_NB = _N // _BN


def _scale_body(xt_ref, deg_ref, xs_ref, dinv_ref):
    dv = 1.0 / jnp.sqrt(deg_ref[...])
    xs_ref[0] = xt_ref[0] * dv
    dinv_ref[...] = dv


_scale_call = pl.pallas_call(
    _scale_body,
    grid=(_P, _NB),
    in_specs=[
        pl.BlockSpec((1, _BN, _F), lambda p, i: (p, i, 0)),
        pl.BlockSpec((_BN, 1), lambda p, i: (i, 0)),
    ],
    out_specs=[
        pl.BlockSpec((1, _BN, _F), lambda p, i: (p, i, 0)),
        pl.BlockSpec((_BN, 1), lambda p, i: (i, 0)),
    ],
    out_shape=[
        jax.ShapeDtypeStruct((_P, _N, _F), jnp.float32),
        jax.ShapeDtypeStruct((_N, 1), jnp.float32),
    ],
    compiler_params=pltpu.CompilerParams(
        dimension_semantics=("arbitrary", "arbitrary")),
)


def _tc_body(y0_ref, y1_ref, xt_ref, dinv_ref, wcz_ref, bcz_ref, wcr_ref,
             bcr_ref, wch_ref, bch_ref, wz_ref, bz_ref, wr_ref, br_ref,
             wh_ref, bh_ref, wfc_ref, bfc_ref, out_ref, h_scr, hsum_scr):
    i = pl.program_id(0)
    p = pl.program_id(1)

    @pl.when(p == 0)
    def _():
        h_scr[...] = jnp.zeros((_BN, _HID), jnp.float32)

    def dot(a, b):
        return lax.dot_general(a, b, (((1,), (0,)), ((), ())),
                               precision=lax.Precision.HIGHEST,
                               preferred_element_type=jnp.float32)

    dv = dinv_ref[...]
    y = (y0_ref[0] + y1_ref[0]) * dv + xt_ref[0] * (dv * dv)
    cz = dot(y, wcz_ref[...]) + bcz_ref[...]
    cr = dot(y, wcr_ref[...]) + bcr_ref[...]
    ch = dot(y, wch_ref[...]) + bch_ref[...]
    h = h_scr[...]
    z = jax.nn.sigmoid(dot(cz, wz_ref[0:_HID, :])
                       + dot(h, wz_ref[_HID:, :]) + bz_ref[...])
    r = jax.nn.sigmoid(dot(cr, wr_ref[0:_HID, :])
                       + dot(h, wr_ref[_HID:, :]) + br_ref[...])
    ht = jnp.tanh(dot(ch, wh_ref[0:_HID, :])
                  + dot(h * r, wh_ref[_HID:, :]) + bh_ref[...])
    hn = z * h + (1.0 - z) * ht
    h_scr[...] = hn

    @pl.when(p == _P - 1)
    def _():
        @pl.when(i == 0)
        def _():
            hsum_scr[...] = jnp.zeros((1, _HID), jnp.float32)

        hsum_scr[...] = hsum_scr[...] + jnp.sum(hn, axis=0, keepdims=True)

        @pl.when(i == _NB - 1)
        def _():
            hm = jnp.maximum(hsum_scr[...] / jnp.float32(_N), 0.0)
            out_ref[...] = dot(hm, wfc_ref[...]) + bfc_ref[...]


def _full(shape):
    return pl.BlockSpec(shape, lambda i, p: (0,) * len(shape))


_tc_call = pl.pallas_call(
    _tc_body,
    grid=(_NB, _P),
    in_specs=[
        pl.BlockSpec((1, _BN, _F), lambda i, p: (p, i, 0)),
        pl.BlockSpec((1, _BN, _F), lambda i, p: (p, i, 0)),
        pl.BlockSpec((1, _BN, _F), lambda i, p: (p, i, 0)),
        pl.BlockSpec((_BN, 1), lambda i, p: (i, 0)),
        _full((_F, _HID)), _full((1, _HID)),
        _full((_F, _HID)), _full((1, _HID)),
        _full((_F, _HID)), _full((1, _HID)),
        _full((2 * _HID, _HID)), _full((1, _HID)),
        _full((2 * _HID, _HID)), _full((1, _HID)),
        _full((2 * _HID, _HID)), _full((1, _HID)),
        _full((_HID, 1)), _full((1, 1)),
    ],
    out_specs=pl.BlockSpec((1, 1), lambda i, p: (0, 0)),
    out_shape=jax.ShapeDtypeStruct((1, 1), jnp.float32),
    scratch_shapes=[
        pltpu.VMEM((_BN, _HID), jnp.float32),
        pltpu.VMEM((1, _HID), jnp.float32),
    ],
    compiler_params=pltpu.CompilerParams(
        dimension_semantics=("arbitrary", "arbitrary")),
)


def kernel(X, edge_index, edge_weight, W_cz, b_cz, W_cr, b_cr, W_ch, b_ch,
           Wz, bz, Wr, br, Wh, bh, W_fc, b_fc):
    xt = jnp.transpose(X, (2, 0, 1))
    src = edge_index[0].reshape(_NW, _EPW)
    dst = edge_index[1]
    dstg = dst.reshape(_NW, _NCHUNK, _CHUNK)
    ewg = edge_weight.reshape(_NW, _EPW)
    dstdeg = dst.reshape(_NS, _DEG_B, _DEG_R, _DEG_C)
    ewdeg = edge_weight.reshape(_NS, _DEG_B, _DEG_R, _DEG_C)
    (deg,) = _deg_call(dstdeg, ewdeg)
    xs, dinv = _scale_call(xt, deg[:_N].reshape(_N, 1))
    y0, y1 = _agg_call(xs.reshape(_P * _N, _F), src, dstg, ewg)
    out = _tc_call(y0, y1, xt, dinv,
                   W_cz, b_cz.reshape(1, _HID),
                   W_cr, b_cr.reshape(1, _HID),
                   W_ch, b_ch.reshape(1, _HID),
                   Wz, bz.reshape(1, _HID),
                   Wr, br.reshape(1, _HID),
                   Wh, bh.reshape(1, _HID),
                   W_fc, b_fc.reshape(1, 1))
    return out
